# baseline (speedup 1.0000x reference)
import functools

import jax
import jax.numpy as jnp
from jax import lax
from jax.experimental import pallas as pl
from jax.experimental.pallas import tpu as pltpu
from jax.experimental.pallas import tpu_sc as plsc

N_NODES = 10000
N_EDGES = 320000
DIN = 128
C = 16

NC = 2
NS = 16
NW = NC * NS
EW = N_EDGES // NW
K = 1000
NCH = EW // K

_SC_MESH = dict(core_axis_name="c", subcore_axis_name="s",
                num_cores=NC, num_subcores=NS)


def _prologue_body(x_ref, w_ref, b_ref, t_ref, logb0_ref, h_ref, bm_ref):
    z = jnp.dot(x_ref[...], w_ref[...],
                preferred_element_type=jnp.float32) + b_ref[...]
    m = jnp.max(z, axis=-1, keepdims=True)
    lse = jnp.log(jnp.sum(jnp.exp(z - m), axis=-1, keepdims=True)) + m
    logb0_ref[...] = z - lse

    @pl.when(pl.program_id(0) == 0)
    def _():
        t = t_ref[...]
        s = jnp.sum(t * t, axis=1, keepdims=True)
        logh = -(s + s.T - 2.0 * jnp.dot(t, t.T,
                                         preferred_element_type=jnp.float32))
        h = jnp.exp(logh)
        hrow = jnp.concatenate([h] * 8, axis=1)
        htile = jnp.concatenate([hrow] * 8, axis=0)
        ii = lax.broadcasted_iota(jnp.int32, (8 * C, 8 * C), 0) // C
        jj = lax.broadcasted_iota(jnp.int32, (8 * C, 8 * C), 1) // C
        blk = (ii == jj)
        h_ref[...] = jnp.where(blk, htile, 0.0)
        bm_ref[...] = jnp.where(blk, 1.0, 0.0)


def _prologue(x, w, b2, t):
    rb = 2000
    grid = (N_NODES // rb,)
    return pl.pallas_call(
        _prologue_body,
        grid=grid,
        in_specs=[
            pl.BlockSpec((rb, DIN), lambda i: (i, 0)),
            pl.BlockSpec((DIN, C), lambda i: (0, 0)),
            pl.BlockSpec((1, C), lambda i: (0, 0)),
            pl.BlockSpec((C, C), lambda i: (0, 0)),
        ],
        out_specs=[
            pl.BlockSpec((rb, C), lambda i: (i, 0)),
            pl.BlockSpec((8 * C, 8 * C), lambda i: (0, 0)),
            pl.BlockSpec((8 * C, 8 * C), lambda i: (0, 0)),
        ],
        out_shape=[
            jax.ShapeDtypeStruct((N_NODES, C), jnp.float32),
            jax.ShapeDtypeStruct((8 * C, 8 * C), jnp.float32),
            jax.ShapeDtypeStruct((8 * C, 8 * C), jnp.float32),
        ],
    )(x, w, b2, t)


def _group_max(a):
    ms = [jnp.max(a[:, k * C:(k + 1) * C], axis=1, keepdims=True)
          for k in range(8)]
    return jnp.concatenate(
        [jnp.broadcast_to(mk, (a.shape[0], C)) for mk in ms], axis=1)


def _msg_from(a, hb_ref, bm_ref, out_ref):
    p = jnp.exp(a - 44.0)
    q = jnp.dot(p, hb_ref[...], preferred_element_type=jnp.float32)
    q = jnp.maximum(q, 1e-37)
    z = jnp.dot(q, bm_ref[...], preferred_element_type=jnp.float32)
    out_ref[...] = jnp.log(q / z)


def _edge_body2(g1_ref, g2_ref, hb_ref, bm_ref, out_ref):
    _msg_from(g1_ref[...] - g2_ref[...], hb_ref, bm_ref, out_ref)


def _edge_body1(g1_ref, hb_ref, bm_ref, out_ref):
    _msg_from(g1_ref[...], hb_ref, bm_ref, out_ref)


EP = N_EDGES // 8
NP = N_NODES // 8


def _edge_update(g1p, g2p, hb, bm):
    rb = 2000
    grid = (EP // rb,)
    espec = pl.BlockSpec((rb, 8 * C), lambda i: (i, 0))
    hspec = pl.BlockSpec((8 * C, 8 * C), lambda i: (0, 0))
    if g2p is None:
        return pl.pallas_call(
            _edge_body1, grid=grid,
            in_specs=[espec, hspec, hspec],
            out_specs=espec,
            out_shape=jax.ShapeDtypeStruct((EP, 8 * C), jnp.float32),
        )(g1p, hb, bm)
    return pl.pallas_call(
        _edge_body2, grid=grid,
        in_specs=[espec, espec, hspec, hspec],
        out_specs=espec,
        out_shape=jax.ShapeDtypeStruct((EP, 8 * C), jnp.float32),
    )(g1p, g2p, hb, bm)


def _combine_body(a0_ref, a1_ref, logb0_ref, bm_ref, out_ref):
    s = a0_ref[0] + a1_ref[0] + logb0_ref[...]
    m = _group_max(s)
    e = jnp.exp(s - m)
    z = jnp.dot(e, bm_ref[...], preferred_element_type=jnp.float32)
    out_ref[...] = s - m - jnp.log(z)


def _combine(agg3p, logb0p, bm):
    spec0 = pl.BlockSpec((1, NP, 8 * C), lambda i: (0, 0, 0))
    spec1 = pl.BlockSpec((1, NP, 8 * C), lambda i: (1, 0, 0))
    spec = pl.BlockSpec((NP, 8 * C), lambda i: (0, 0))
    hspec = pl.BlockSpec((8 * C, 8 * C), lambda i: (0, 0))
    return pl.pallas_call(
        _combine_body, grid=(1,),
        in_specs=[spec0, spec1, spec, hspec],
        out_specs=spec,
        out_shape=jax.ShapeDtypeStruct((NP, 8 * C), jnp.float32),
    )(agg3p, agg3p, logb0p, bm)


def _gather_pipeline(tables, idx_vs, outs, bufs, gsems, ssems, base):
    nt = len(tables)

    def issue(j, s):
        for t in range(nt):
            pltpu.async_copy(tables[t].at[idx_vs[t].at[j]], bufs[t][s],
                             gsems[t][s])

    def wait_g(s):
        for t in range(nt):
            pltpu.make_async_copy(tables[t].at[idx_vs[t].at[0]], bufs[t][s],
                                  gsems[t][s]).wait()

    def store(j, s):
        for t in range(nt):
            pltpu.async_copy(bufs[t][s], outs[t].at[pl.ds(base + j * K, K)],
                             ssems[t][s])

    def wait_st(s):
        for t in range(nt):
            pltpu.make_async_copy(bufs[t][s], outs[t].at[pl.ds(base, K)],
                                  ssems[t][s]).wait()

    issue(0, 0)
    issue(1, 1)

    def pair(g, c):
        j0 = 2 * g
        j1 = j0 + 1
        wait_g(0)
        store(j0, 0)
        wait_g(1)
        store(j1, 1)
        wait_st(0)

        @pl.when(j0 + 2 < NCH)
        def _():
            issue(j0 + 2, 0)

        wait_st(1)

        @pl.when(j1 + 2 < NCH)
        def _():
            issue(j1 + 2, 1)

        return c

    lax.fori_loop(0, NCH // 2, pair, 0)
    if NCH % 2 == 1:
        wait_g(0)
        store(NCH - 1, 0)
        wait_st(0)


def _gather2_body(logb_hbm, src_hbm, msg_hbm, rv_hbm, g1_hbm, g2_hbm,
                  idx1_v, idx2_v, b1a, b1b, b2a, b2b,
                  g1s, g2s, g3s, g4s, s1s, s2s, s3s, s4s):
    cid = lax.axis_index("c")
    sid = lax.axis_index("s")
    w = cid * NS + sid
    base = w * EW
    pltpu.sync_copy(src_hbm.at[w], idx1_v)
    pltpu.sync_copy(rv_hbm.at[w], idx2_v)
    _gather_pipeline(
        (logb_hbm, msg_hbm), (idx1_v, idx2_v), (g1_hbm, g2_hbm),
        ((b1a, b1b), (b2a, b2b)),
        ((g1s, g2s), (g3s, g4s)), ((s1s, s2s), (s3s, s4s)), base)


def _gather1_body(logb_hbm, src_hbm, g1_hbm,
                  idx1_v, b1a, b1b, g1s, g2s, s1s, s2s):
    cid = lax.axis_index("c")
    sid = lax.axis_index("s")
    w = cid * NS + sid
    base = w * EW
    pltpu.sync_copy(src_hbm.at[w], idx1_v)
    _gather_pipeline((logb_hbm,), (idx1_v,), (g1_hbm,),
                     ((b1a, b1b),), ((g1s, g2s),), ((s1s, s2s),), base)


def _scatter_body(msg_hbm, dst_hbm, zeros_hbm, agg_hbm, idx_v, rows_a,
                  rows_b, shared, sem_a, sem_b, sem_c, sem_d):
    cid = lax.axis_index("c")
    sid = lax.axis_index("s")
    w = cid * NS + sid
    base = w * EW

    @pl.when(sid == 0)
    def _():
        pltpu.sync_copy(zeros_hbm, shared)

    plsc.subcore_barrier()
    pltpu.sync_copy(dst_hbm.at[w], idx_v)
    rbufs = (rows_a, rows_b)
    lsems = (sem_a, sem_b)

    def load(j, s):
        pltpu.async_copy(msg_hbm.at[pl.ds(base + j * K, K)], rbufs[s],
                         lsems[s])

    def wait_l(s):
        pltpu.make_async_copy(msg_hbm.at[pl.ds(base, K)], rbufs[s],
                              lsems[s]).wait()

    ssems = (sem_c, sem_d)

    def scat(j, s):
        pltpu.async_copy(rbufs[s], shared.at[idx_v.at[j]], ssems[s], add=True)

    def wait_scat(s):
        pltpu.make_async_copy(rbufs[s], shared.at[idx_v.at[0]],
                              ssems[s]).wait()

    load(0, 0)
    load(1, 1)

    def pair(g, c):
        j0 = 2 * g
        j1 = j0 + 1
        wait_l(0)
        scat(j0, 0)
        wait_l(1)
        scat(j1, 1)
        wait_scat(0)

        @pl.when(j0 + 2 < NCH)
        def _():
            load(j0 + 2, 0)

        wait_scat(1)

        @pl.when(j1 + 2 < NCH)
        def _():
            load(j1 + 2, 1)

        return c

    lax.fori_loop(0, NCH // 2, pair, 0)
    if NCH % 2 == 1:
        wait_l(0)
        scat(NCH - 1, 0)
        wait_scat(0)
    plsc.subcore_barrier()

    @pl.when(sid == 0)
    def _():
        pltpu.sync_copy(shared, agg_hbm.at[pl.ds(cid * N_NODES, N_NODES)])


@functools.lru_cache(maxsize=1)
def _sc_kernels():
    mesh = plsc.VectorSubcoreMesh(**_SC_MESH)
    cparams = pltpu.CompilerParams(use_tc_tiling_on_sc=False)
    e_out = jax.ShapeDtypeStruct((N_EDGES, C), jnp.float32)
    gather2 = pl.kernel(
        _gather2_body,
        out_type=[e_out, e_out],
        mesh=mesh,
        compiler_params=cparams,
        scratch_types=(
            [pltpu.VMEM((NCH, K), jnp.int32)] * 2
            + [pltpu.VMEM((K, C), jnp.float32)] * 4
            + [pltpu.SemaphoreType.DMA] * 8
        ),
    )
    gather1 = pl.kernel(
        _gather1_body,
        out_type=e_out,
        mesh=mesh,
        compiler_params=cparams,
        scratch_types=(
            [pltpu.VMEM((NCH, K), jnp.int32)]
            + [pltpu.VMEM((K, C), jnp.float32)] * 2
            + [pltpu.SemaphoreType.DMA] * 4
        ),
    )
    scatter = pl.kernel(
        _scatter_body,
        out_type=jax.ShapeDtypeStruct((NC * N_NODES, C), jnp.float32),
        mesh=mesh,
        compiler_params=cparams,
        scratch_types=(
            [pltpu.VMEM((NCH, K), jnp.int32)]
            + [pltpu.VMEM((K, C), jnp.float32)] * 2
            + [pltpu.VMEM_SHARED((N_NODES, C), jnp.float32)]
            + [pltpu.SemaphoreType.DMA] * 4
        ),
    )
    return gather1, gather2, scatter


def kernel(x, edge_index, rv, W, b, T):
    eidx = edge_index.reshape(2 * NW, NCH, K)
    src2 = eidx[:NW]
    dst2 = eidx[NW:]
    rv2 = rv.reshape(NW, NCH, K)
    zeros = jnp.zeros((N_NODES, C), jnp.float32)
    gather1, gather2, scatter = _sc_kernels()

    logb0, hb, bm = _prologue(x, W, b.reshape(1, C), T)
    logb0p = jnp.reshape(logb0, (NP, 8 * C))

    g1 = gather1(logb0, src2)
    msgp = _edge_update(jnp.reshape(g1, (EP, 8 * C)), None, hb, bm)
    msg = jnp.reshape(msgp, (N_EDGES, C))
    agg2 = scatter(msg, dst2, zeros)
    logbp = _combine(jnp.reshape(agg2, (2, NP, 8 * C)), logb0p, bm)

    for _ in range(4):
        logb = jnp.reshape(logbp, (N_NODES, C))
        g1, g2 = gather2(logb, src2, msg, rv2)
        msgp = _edge_update(jnp.reshape(g1, (EP, 8 * C)),
                            jnp.reshape(g2, (EP, 8 * C)), hb, bm)
        msg = jnp.reshape(msgp, (N_EDGES, C))
        agg2 = scatter(msg, dst2, zeros)
        logbp = _combine(jnp.reshape(agg2, (2, NP, 8 * C)), logb0p, bm)

    return jnp.reshape(logbp, (N_NODES, C))


# speedup vs baseline: 1.0565x; 1.0565x over previous
import functools

import jax
import jax.numpy as jnp
from jax import lax
from jax.experimental import pallas as pl
from jax.experimental.pallas import tpu as pltpu
from jax.experimental.pallas import tpu_sc as plsc

N_NODES = 10000
N_EDGES = 320000
DIN = 128
C = 16

NC = 2
NS = 16
NW = NC * NS
EW = N_EDGES // NW
K = 1000
NCH = EW // K

_SC_MESH = dict(core_axis_name="c", subcore_axis_name="s",
                num_cores=NC, num_subcores=NS)


def _prologue_body(x_ref, w_ref, b_ref, t_ref, logb0_ref, h_ref, bm_ref):
    z = jnp.dot(x_ref[...], w_ref[...],
                preferred_element_type=jnp.float32) + b_ref[...]
    m = jnp.max(z, axis=-1, keepdims=True)
    lse = jnp.log(jnp.sum(jnp.exp(z - m), axis=-1, keepdims=True)) + m
    logb0_ref[...] = z - lse

    @pl.when(pl.program_id(0) == 0)
    def _():
        t = t_ref[...]
        s = jnp.sum(t * t, axis=1, keepdims=True)
        logh = -(s + s.T - 2.0 * jnp.dot(t, t.T,
                                         preferred_element_type=jnp.float32))
        h = jnp.exp(logh)
        hrow = jnp.concatenate([h] * 8, axis=1)
        htile = jnp.concatenate([hrow] * 8, axis=0)
        ii = lax.broadcasted_iota(jnp.int32, (8 * C, 8 * C), 0) // C
        jj = lax.broadcasted_iota(jnp.int32, (8 * C, 8 * C), 1) // C
        blk = (ii == jj)
        h_ref[...] = jnp.where(blk, htile, 0.0)
        bm_ref[...] = jnp.where(blk, 1.0, 0.0)


def _prologue(x, w, b2, t):
    rb = 2000
    grid = (N_NODES // rb,)
    return pl.pallas_call(
        _prologue_body,
        grid=grid,
        in_specs=[
            pl.BlockSpec((rb, DIN), lambda i: (i, 0)),
            pl.BlockSpec((DIN, C), lambda i: (0, 0)),
            pl.BlockSpec((1, C), lambda i: (0, 0)),
            pl.BlockSpec((C, C), lambda i: (0, 0)),
        ],
        out_specs=[
            pl.BlockSpec((rb, C), lambda i: (i, 0)),
            pl.BlockSpec((8 * C, 8 * C), lambda i: (0, 0)),
            pl.BlockSpec((8 * C, 8 * C), lambda i: (0, 0)),
        ],
        out_shape=[
            jax.ShapeDtypeStruct((N_NODES, C), jnp.float32),
            jax.ShapeDtypeStruct((8 * C, 8 * C), jnp.float32),
            jax.ShapeDtypeStruct((8 * C, 8 * C), jnp.float32),
        ],
    )(x, w, b2, t)


def _group_max(a):
    ms = [jnp.max(a[:, k * C:(k + 1) * C], axis=1, keepdims=True)
          for k in range(8)]
    return jnp.concatenate(
        [jnp.broadcast_to(mk, (a.shape[0], C)) for mk in ms], axis=1)


def _msg_from(a, hb_ref, bm_ref, out_ref):
    p = jnp.exp(a - 44.0)
    q = jnp.dot(p, hb_ref[...], preferred_element_type=jnp.float32)
    q = jnp.maximum(q, 1e-37)
    z = jnp.dot(q, bm_ref[...], preferred_element_type=jnp.float32)
    out_ref[...] = jnp.log(q / z)


def _edge_body2(g1_ref, g2_ref, hb_ref, bm_ref, out_ref):
    _msg_from(g1_ref[...] - g2_ref[...], hb_ref, bm_ref, out_ref)


def _edge_body1(g1_ref, hb_ref, bm_ref, out_ref):
    _msg_from(g1_ref[...], hb_ref, bm_ref, out_ref)


EP = N_EDGES // 8
NP = N_NODES // 8


def _edge_update(g1p, g2p, hb, bm):
    rb = 4000
    grid = (EP // rb,)
    espec = pl.BlockSpec((rb, 8 * C), lambda i: (i, 0))
    hspec = pl.BlockSpec((8 * C, 8 * C), lambda i: (0, 0))
    if g2p is None:
        return pl.pallas_call(
            _edge_body1, grid=grid,
            in_specs=[espec, hspec, hspec],
            out_specs=espec,
            out_shape=jax.ShapeDtypeStruct((EP, 8 * C), jnp.float32),
        )(g1p, hb, bm)
    return pl.pallas_call(
        _edge_body2, grid=grid,
        in_specs=[espec, espec, hspec, hspec],
        out_specs=espec,
        out_shape=jax.ShapeDtypeStruct((EP, 8 * C), jnp.float32),
    )(g1p, g2p, hb, bm)


def _combine_body(a0_ref, a1_ref, logb0_ref, bm_ref, out_ref):
    s = a0_ref[0] + a1_ref[0] + logb0_ref[...]
    m = _group_max(s)
    e = jnp.exp(s - m)
    z = jnp.dot(e, bm_ref[...], preferred_element_type=jnp.float32)
    out_ref[...] = s - m - jnp.log(z)


def _combine(agg3p, logb0p, bm):
    spec0 = pl.BlockSpec((1, NP, 8 * C), lambda i: (0, 0, 0))
    spec1 = pl.BlockSpec((1, NP, 8 * C), lambda i: (1, 0, 0))
    spec = pl.BlockSpec((NP, 8 * C), lambda i: (0, 0))
    hspec = pl.BlockSpec((8 * C, 8 * C), lambda i: (0, 0))
    return pl.pallas_call(
        _combine_body, grid=(1,),
        in_specs=[spec0, spec1, spec, hspec],
        out_specs=spec,
        out_shape=jax.ShapeDtypeStruct((NP, 8 * C), jnp.float32),
    )(agg3p, agg3p, logb0p, bm)


def _gather_pipeline(tables, idx_vs, outs, bufs, gsems, ssems, base):
    nt = len(tables)

    def issue(j, s):
        for t in range(nt):
            pltpu.async_copy(tables[t].at[idx_vs[t].at[j]], bufs[t][s],
                             gsems[t][s])

    def wait_g(s):
        for t in range(nt):
            pltpu.make_async_copy(tables[t].at[idx_vs[t].at[0]], bufs[t][s],
                                  gsems[t][s]).wait()

    def store(j, s):
        for t in range(nt):
            pltpu.async_copy(bufs[t][s], outs[t].at[pl.ds(base + j * K, K)],
                             ssems[t][s])

    def wait_st(s):
        for t in range(nt):
            pltpu.make_async_copy(bufs[t][s], outs[t].at[pl.ds(base, K)],
                                  ssems[t][s]).wait()

    issue(0, 0)
    issue(1, 1)

    def pair(g, c):
        j0 = 2 * g
        j1 = j0 + 1
        wait_g(0)
        store(j0, 0)
        wait_g(1)
        store(j1, 1)
        wait_st(0)

        @pl.when(j0 + 2 < NCH)
        def _():
            issue(j0 + 2, 0)

        wait_st(1)

        @pl.when(j1 + 2 < NCH)
        def _():
            issue(j1 + 2, 1)

        return c

    lax.fori_loop(0, NCH // 2, pair, 0)
    if NCH % 2 == 1:
        wait_g(0)
        store(NCH - 1, 0)
        wait_st(0)


def _gather2_body(logb_hbm, src_hbm, msg_hbm, rv_hbm, g1_hbm, g2_hbm,
                  idx1_v, idx2_v, b1a, b1b, b2a, b2b,
                  g1s, g2s, g3s, g4s, s1s, s2s, s3s, s4s):
    cid = lax.axis_index("c")
    sid = lax.axis_index("s")
    w = cid * NS + sid
    base = w * EW
    pltpu.sync_copy(src_hbm.at[w], idx1_v)
    pltpu.sync_copy(rv_hbm.at[w], idx2_v)
    _gather_pipeline(
        (logb_hbm, msg_hbm), (idx1_v, idx2_v), (g1_hbm, g2_hbm),
        ((b1a, b1b), (b2a, b2b)),
        ((g1s, g2s), (g3s, g4s)), ((s1s, s2s), (s3s, s4s)), base)


def _gather1_body(logb_hbm, src_hbm, g1_hbm,
                  idx1_v, b1a, b1b, g1s, g2s, s1s, s2s):
    cid = lax.axis_index("c")
    sid = lax.axis_index("s")
    w = cid * NS + sid
    base = w * EW
    pltpu.sync_copy(src_hbm.at[w], idx1_v)
    _gather_pipeline((logb_hbm,), (idx1_v,), (g1_hbm,),
                     ((b1a, b1b),), ((g1s, g2s),), ((s1s, s2s),), base)


def _scatter_body(msg_hbm, dst_hbm, zeros_hbm, agg_hbm, idx_v, rows_a,
                  rows_b, shared, sem_a, sem_b, sem_c, sem_d):
    cid = lax.axis_index("c")
    sid = lax.axis_index("s")
    w = cid * NS + sid
    base = w * EW

    @pl.when(sid == 0)
    def _():
        pltpu.sync_copy(zeros_hbm, shared)

    plsc.subcore_barrier()
    pltpu.sync_copy(dst_hbm.at[w], idx_v)
    rbufs = (rows_a, rows_b)
    lsems = (sem_a, sem_b)

    def load(j, s):
        pltpu.async_copy(msg_hbm.at[pl.ds(base + j * K, K)], rbufs[s],
                         lsems[s])

    def wait_l(s):
        pltpu.make_async_copy(msg_hbm.at[pl.ds(base, K)], rbufs[s],
                              lsems[s]).wait()

    ssems = (sem_c, sem_d)

    def scat(j, s):
        pltpu.async_copy(rbufs[s], shared.at[idx_v.at[j]], ssems[s], add=True)

    def wait_scat(s):
        pltpu.make_async_copy(rbufs[s], shared.at[idx_v.at[0]],
                              ssems[s]).wait()

    load(0, 0)
    load(1, 1)

    def pair(g, c):
        j0 = 2 * g
        j1 = j0 + 1
        wait_l(0)
        scat(j0, 0)
        wait_l(1)
        scat(j1, 1)
        wait_scat(0)

        @pl.when(j0 + 2 < NCH)
        def _():
            load(j0 + 2, 0)

        wait_scat(1)

        @pl.when(j1 + 2 < NCH)
        def _():
            load(j1 + 2, 1)

        return c

    lax.fori_loop(0, NCH // 2, pair, 0)
    if NCH % 2 == 1:
        wait_l(0)
        scat(NCH - 1, 0)
        wait_scat(0)
    plsc.subcore_barrier()

    @pl.when(sid == 0)
    def _():
        pltpu.sync_copy(shared, agg_hbm.at[pl.ds(cid * N_NODES, N_NODES)])


@functools.lru_cache(maxsize=1)
def _sc_kernels():
    mesh = plsc.VectorSubcoreMesh(**_SC_MESH)
    cparams = pltpu.CompilerParams(use_tc_tiling_on_sc=False)
    e_out = jax.ShapeDtypeStruct((N_EDGES, C), jnp.float32)
    gather2 = pl.kernel(
        _gather2_body,
        out_type=[e_out, e_out],
        mesh=mesh,
        compiler_params=cparams,
        scratch_types=(
            [pltpu.VMEM((NCH, K), jnp.int32)] * 2
            + [pltpu.VMEM((K, C), jnp.float32)] * 4
            + [pltpu.SemaphoreType.DMA] * 8
        ),
    )
    gather1 = pl.kernel(
        _gather1_body,
        out_type=e_out,
        mesh=mesh,
        compiler_params=cparams,
        scratch_types=(
            [pltpu.VMEM((NCH, K), jnp.int32)]
            + [pltpu.VMEM((K, C), jnp.float32)] * 2
            + [pltpu.SemaphoreType.DMA] * 4
        ),
    )
    scatter = pl.kernel(
        _scatter_body,
        out_type=jax.ShapeDtypeStruct((NC * N_NODES, C), jnp.float32),
        mesh=mesh,
        compiler_params=cparams,
        scratch_types=(
            [pltpu.VMEM((NCH, K), jnp.int32)]
            + [pltpu.VMEM((K, C), jnp.float32)] * 2
            + [pltpu.VMEM_SHARED((N_NODES, C), jnp.float32)]
            + [pltpu.SemaphoreType.DMA] * 4
        ),
    )
    return gather1, gather2, scatter


def kernel(x, edge_index, rv, W, b, T):
    src2 = edge_index[0].reshape(NW, NCH, K)
    dst2 = edge_index[1].reshape(NW, NCH, K)
    rv2 = rv.reshape(NW, NCH, K)
    zeros = jnp.zeros((N_NODES, C), jnp.float32)
    gather1, gather2, scatter = _sc_kernels()

    logb0, hb, bm = _prologue(x, W, b.reshape(1, C), T)
    logb0p = jnp.reshape(logb0, (NP, 8 * C))

    g1 = gather1(logb0, src2)
    msgp = _edge_update(jnp.reshape(g1, (EP, 8 * C)), None, hb, bm)
    msg = jnp.reshape(msgp, (N_EDGES, C))
    agg2 = scatter(msg, dst2, zeros)
    logbp = _combine(jnp.reshape(agg2, (2, NP, 8 * C)), logb0p, bm)

    for _ in range(4):
        logb = jnp.reshape(logbp, (N_NODES, C))
        g1, g2 = gather2(logb, src2, msg, rv2)
        msgp = _edge_update(jnp.reshape(g1, (EP, 8 * C)),
                            jnp.reshape(g2, (EP, 8 * C)), hb, bm)
        msg = jnp.reshape(msgp, (N_EDGES, C))
        agg2 = scatter(msg, dst2, zeros)
        logbp = _combine(jnp.reshape(agg2, (2, NP, 8 * C)), logb0p, bm)

    return jnp.reshape(logbp, (N_NODES, C))
